# baseline (speedup 1.0000x reference)
import functools

import jax
import jax.numpy as jnp
from jax import lax
from jax.experimental import pallas as pl
from jax.experimental.pallas import tpu as pltpu
from jax.experimental.pallas import tpu_sc as plsc

F, B, L = 26, 4096, 20
V, D = 100000, 32

NW = 32
BPW = B // NW
NB = 32
NSUB = BPW // NB
ROWS = NB * L
NCH = NSUB * F


def _sc_body(ids_hbm, tab_hbm, out_hbm, idx0, idx1, rows0, rows1, out_v,
             sem_g0, sem_g1, sem_ids):
    cid = lax.axis_index("c")
    sid = lax.axis_index("s")
    wid = sid * 2 + cid

    idx = (idx0, idx1)
    rows = (rows0, rows1)
    semg = (sem_g0, sem_g1)

    def id_offset(c):
        f = c % F
        sub = c // F
        return f * (B * L) + wid * (BPW * L) + sub * (NB * L)

    def start_ids(c, p):
        pltpu.async_copy(ids_hbm.at[pl.ds(id_offset(c), ROWS)], idx[p],
                         sem_ids)

    def wait_ids(p):
        pltpu.make_async_copy(ids_hbm.at[pl.ds(0, ROWS)], idx[p],
                              sem_ids).wait()

    def fire_gather(c, p):
        pltpu.async_copy(tab_hbm.at[idx[p]], rows[p], semg[p])

    def drain_gather(c, p):
        pltpu.make_async_copy(tab_hbm.at[idx[p]], rows[p], semg[p]).wait()

    def compute(c, p):
        f = c % F
        sub = c // F
        rp = rows[p]

        def bag(b, carry):
            base = b * L
            a0 = rp[base, pl.ds(0, 16)]
            a1 = rp[base, pl.ds(16, 16)]
            for l in range(1, L):
                a0 = a0 + rp[base + l, pl.ds(0, 16)]
                a1 = a1 + rp[base + l, pl.ds(16, 16)]
            out_v[b, pl.ds(f * D, 16)] = a0
            out_v[b, pl.ds(f * D + 16, 16)] = a1
            return carry

        lax.fori_loop(0, NB, bag, 0)

        @pl.when(f == F - 1)
        def _():
            b0 = wid * BPW + sub * NB
            pltpu.sync_copy(out_v, out_hbm.at[pl.ds(b0, NB)])

    pltpu.sync_copy(ids_hbm.at[pl.ds(id_offset(0), ROWS)], idx[0])
    fire_gather(0, 0)
    start_ids(1, 1)

    def pair_body(i, carry):
        for p in (0, 1):
            c = i * 2 + p
            q = 1 - p

            @pl.when(c + 1 < NCH)
            def _():
                wait_ids(q)
                fire_gather(c + 1, q)

            drain_gather(c, p)

            @pl.when(c + 2 < NCH)
            def _():
                start_ids(c + 2, p)

            compute(c, p)
        return carry

    lax.fori_loop(0, NCH // 2, pair_body, 0)


@jax.jit
def _ebc_lookup(ids1d, tflat):
    mesh = plsc.VectorSubcoreMesh(core_axis_name="c", subcore_axis_name="s")
    run = pl.kernel(
        _sc_body,
        out_type=jax.ShapeDtypeStruct((B, F * D), jnp.float32),
        mesh=mesh,
        scratch_types=[
            pltpu.VMEM((ROWS,), jnp.int32),
            pltpu.VMEM((ROWS,), jnp.int32),
            pltpu.VMEM((ROWS, D), jnp.float32),
            pltpu.VMEM((ROWS, D), jnp.float32),
            pltpu.VMEM((NB, F * D), jnp.float32),
            pltpu.SemaphoreType.DMA,
            pltpu.SemaphoreType.DMA,
            pltpu.SemaphoreType.DMA,
        ],
        compiler_params=pltpu.CompilerParams(use_tc_tiling_on_sc=False),
    )
    return run(ids1d, tflat)


def _rows_view(table):
    tp = jax.lax.optimization_barrier(jnp.transpose(table, (0, 2, 1)))
    return jnp.transpose(tp, (0, 2, 1)).reshape(F * V, D)


def kernel(values, table0, table1):
    offs = (jnp.arange(F, dtype=jnp.int32) * V)[:, None, None]
    ids1d = (values.astype(jnp.int32) + offs).reshape(-1)
    out0 = _ebc_lookup(ids1d, _rows_view(table0))
    out1 = _ebc_lookup(ids1d, _rows_view(table1))
    return (out0, out1)
